# Initial kernel scaffold; baseline (speedup 1.0000x reference)
import jax
import jax.numpy as jnp
from jax.experimental import pallas as pl


def kernel(x, edge_index, W, b, beta1, beta2):
    raise NotImplementedError("write your pallas kernel here")


# speedup vs baseline: 27.1438x; 27.1438x over previous
import functools

import jax
import jax.numpy as jnp
from jax import lax
from jax.experimental import pallas as pl
from jax.experimental.pallas import tpu as pltpu
from jax.experimental.pallas import tpu_sc as plsc

NC = 2
NS = 16
LN = 16


def _lin_norm_body(x_ref, w_ref, b_ref, xn_ref, ne_ref):
    x = x_ref[...]
    h = lax.dot_general(x, w_ref[...], (((1,), (1,)), ((), ())),
                        preferred_element_type=jnp.float32)
    h = jnp.maximum(h + b_ref[...], 0.0)
    nrm = jnp.sqrt(jnp.sum(h * h, axis=1, keepdims=True))
    ne = jnp.maximum(nrm, 1e-12)
    xn_ref[...] = h / ne
    ne_ref[...] = ne


def _lin_norm(x, W, b2):
    n, _ = x.shape
    h = W.shape[0]
    return pl.pallas_call(
        _lin_norm_body,
        out_shape=[jax.ShapeDtypeStruct((n, h), jnp.float32),
                   jax.ShapeDtypeStruct((n, 1), jnp.float32)],
    )(x, W, b2)


def _combine_body(eps, n, np_ref, dp_ref, xn_ref, ne_ref, beta_ref,
                  xo_ref, no_ref):
    xn = xn_ref[...]
    ne = ne_ref[...]
    beta = beta_ref[...]
    s2 = jnp.sum(xn * xn, axis=1, keepdims=True)
    selfw = jnp.exp(beta * s2)
    h = xn.shape[1]
    nsum = np_ref[0] + np_ref[1]
    numer = lax.slice(nsum, (0, 0), (n, h)) + selfw * (xn * ne)
    dsum = dp_ref[0] + dp_ref[1]
    dn = lax.slice(dsum, (0, 0), (n, 1)) + selfw
    out = numer / (dn + 1e-16)
    nrm = jnp.sqrt(jnp.sum(out * out, axis=1, keepdims=True))
    xo_ref[...] = out / jnp.maximum(nrm, eps)
    no_ref[...] = jnp.maximum(nrm, 1e-12)


def _combine(npart, dpart3, xn, ne, b11, eps):
    n, h = xn.shape
    return pl.pallas_call(
        functools.partial(_combine_body, eps, n),
        out_shape=[jax.ShapeDtypeStruct((n, h), jnp.float32),
                   jax.ShapeDtypeStruct((n, 1), jnp.float32)],
    )(npart, dpart3, xn, ne, b11)


def _make_edge_pass(n, e, h):
    ept = e // (NC * NS)
    blk = 2000
    nblk = ept // blk
    chunk = 80
    nch = blk // chunk
    npad = -(-n // (NS * 128)) * (NS * 128)
    rows = npad // NS
    drows = npad // NS
    assert ept * NC * NS == e and nblk * blk == ept and nch * chunk == blk
    assert h == LN and rows <= blk

    mesh = plsc.VectorSubcoreMesh(core_axis_name="c", subcore_axis_name="s")

    @functools.partial(
        pl.kernel,
        out_type=[jax.ShapeDtypeStruct((NC, npad, h), jnp.float32),
                  jax.ShapeDtypeStruct((NC * npad,), jnp.float32)],
        mesh=mesh,
        compiler_params=pltpu.CompilerParams(needs_layout_passes=False,
                                             use_tc_tiling_on_sc=False),
        scratch_types=[
            pltpu.VMEM((blk,), jnp.int32),
            pltpu.VMEM((blk,), jnp.int32),
            pltpu.VMEM((nch, chunk), jnp.int32),
            pltpu.VMEM((blk, h), jnp.float32),
            pltpu.VMEM((blk, h), jnp.float32),
            pltpu.VMEM((blk, h), jnp.float32),
            pltpu.VMEM((blk,), jnp.float32),
            pltpu.VMEM((n,), jnp.float32),
            pltpu.VMEM((LN,), jnp.float32),
            pltpu.VMEM_SHARED((npad, h), jnp.float32),
            pltpu.VMEM_SHARED((npad,), jnp.float32),
            pltpu.SemaphoreType.DMA,
        ],
    )
    def edge_pass(xn_hbm, ne_hbm, src_hbm, dst_hbm, beta_hbm,
                  np_hbm, dp_hbm,
                  srcv, dstv, dsti, xs, xd, msg, abuf, nrmv, betav,
                  acc_n, acc_d, sem):
        c = lax.axis_index("c")
        s = lax.axis_index("s")
        wid = c * NS + s

        pltpu.sync_copy(ne_hbm, nrmv)
        pltpu.sync_copy(beta_hbm, betav)
        def _zrow(i, carry):
            msg[i, :] = jnp.zeros((LN,), jnp.float32)
            return carry
        lax.fori_loop(0, rows, _zrow, 0)
        pltpu.sync_copy(msg.at[pl.ds(0, rows)],
                        acc_n.at[pl.ds(s * rows, rows)])

        def _zvec(i, carry):
            abuf[pl.ds(i * LN, LN)] = jnp.zeros((LN,), jnp.float32)
            return carry
        lax.fori_loop(0, drows // LN, _zvec, 0)
        pltpu.sync_copy(abuf.at[pl.ds(0, drows)],
                        acc_d.at[pl.ds(s * drows, drows)])
        plsc.subcore_barrier()

        beta_v = betav[...]
        iot = lax.iota(jnp.int32, LN)
        e0 = wid * ept

        def _block(b, carry):
            off = e0 + b * blk
            pltpu.sync_copy(src_hbm.at[pl.ds(off, blk)], srcv)
            pltpu.sync_copy(dst_hbm.at[pl.ds(off, blk)], dstv)
            def _ccopy(k, carry2):
                j = k // (chunk // LN)
                q = k % (chunk // LN)
                dsti[j, pl.ds(q * LN, LN)] = dstv[pl.ds(j * chunk + q * LN, LN)]
                return carry2
            lax.fori_loop(0, nch * (chunk // LN), _ccopy, 0)
            cp1 = pltpu.async_copy(xn_hbm.at[srcv], xs, sem)
            cp2 = pltpu.async_copy(xn_hbm.at[dstv], xd, sem)
            cp1.wait()
            cp2.wait()

            def _group(g, carry2):
                ev = g * LN + iot
                acc = jnp.zeros((LN,), jnp.float32)
                for d in range(h):
                    dv = jnp.full((LN,), d, jnp.int32)
                    acc = acc + (plsc.load_gather(xs, [ev, dv]) *
                                 plsc.load_gather(xd, [ev, dv]))
                a = jnp.exp(beta_v * acc)
                sg = srcv[pl.ds(g * LN, LN)]
                w = a * plsc.load_gather(nrmv, [sg])
                abuf[pl.ds(g * LN, LN)] = a
                for d in range(h):
                    dv = jnp.full((LN,), d, jnp.int32)
                    plsc.store_scatter(msg, [ev, dv],
                                       plsc.load_gather(xs, [ev, dv]) * w)
                return carry2
            lax.fori_loop(0, blk // LN, _group, 0)

            for j in range(nch):
                pltpu.sync_copy(msg.at[pl.ds(j * chunk, chunk)],
                                acc_n.at[dsti.at[j]], add=True)
                pltpu.sync_copy(abuf.at[pl.ds(j * chunk, chunk)],
                                acc_d.at[dsti.at[j]], add=True)
            return carry
        lax.fori_loop(0, nblk, _block, 0)

        plsc.subcore_barrier()
        pltpu.sync_copy(acc_n.at[pl.ds(s * rows, rows)],
                        np_hbm.at[c, pl.ds(s * rows, rows)])
        pltpu.sync_copy(acc_d.at[pl.ds(s * drows, drows)],
                        dp_hbm.at[pl.ds(c * npad + s * drows, drows)])

    return edge_pass, npad


def kernel(x, edge_index, W, b, beta1, beta2):
    n, d = x.shape
    h = W.shape[0]
    e = edge_index.shape[1]
    src = edge_index[0].astype(jnp.int32)
    dst = edge_index[1].astype(jnp.int32)

    edge_pass, npad = _make_edge_pass(n, e, h)

    xn, ne = _lin_norm(x, W, b.reshape(1, h))

    def layer(xn, ne, beta, eps):
        bv = jnp.broadcast_to(beta.astype(jnp.float32).reshape(1), (LN,))
        npart, dpart = edge_pass(xn, ne.reshape(n), src, dst, bv)
        b11 = beta.astype(jnp.float32).reshape(1, 1)
        return _combine(npart, dpart.reshape(NC, npad, 1), xn, ne, b11, eps)

    xn, ne = layer(xn, ne, beta1, 1e-12)
    out, _ = layer(xn, ne, beta2, 5e-4)
    return out
